# baseline (speedup 1.0000x reference)
import functools

import jax
import jax.numpy as jnp
from jax import lax
from jax.experimental import pallas as pl
from jax.experimental.pallas import tpu as pltpu
from jax.experimental.pallas import tpu_sc as plsc

N = 10000
NP = 10240
E = 320000
D = 128
VOCAB = 100010
NG = 512
BLK = 256
NB = NP // BLK
NC = 2
NS = 16
NW = NC * NS
ECH = 128
CPB = 8
NBLK0 = 19
NBLK1 = 1
NCH0 = CPB * NBLK0
NCH1 = CPB * NBLK1
EPAD = ECH * CPB * (NBLK0 + NBLK1) * NS
ROWS_PT = NP // NS


@functools.cache
def _sc_embed_kernel():
    mesh = plsc.VectorSubcoreMesh(core_axis_name="c", subcore_axis_name="s")

    @functools.partial(
        pl.kernel, mesh=mesh,
        compiler_params=pltpu.CompilerParams(needs_layout_passes=False),
        out_type=jax.ShapeDtypeStruct((NP, D), jnp.float32),
        scratch_types=[
            pltpu.VMEM((1, 64), jnp.int32),
            pltpu.VMEM((64, D), jnp.float32),
            pltpu.SemaphoreType.DMA,
        ],
    )
    def body(emb_hbm, xi_hbm, out_hbm, eidx, erows, sem):
        cid = lax.axis_index("c")
        sid = lax.axis_index("s")
        wid = sid * NC + cid
        def chunk(t, carry):
            base = wid * (NP // NW) + t * 64
            pltpu.sync_copy(xi_hbm.at[pl.ds(base, 64)], eidx.at[0])
            pltpu.async_copy(emb_hbm.at[eidx.at[0]], erows, sem).wait()
            pltpu.sync_copy(erows, out_hbm.at[pl.ds(base, 64)])
            return carry
        lax.fori_loop(0, (NP // NW) // 64, chunk, 0)

    return body


def _sc_embed(emb, xi):
    return _sc_embed_kernel()(emb, xi)


@functools.cache
def _sc_sage_kernel():
    mesh = plsc.VectorSubcoreMesh(core_axis_name="c", subcore_axis_name="s")

    @functools.partial(
        pl.kernel, mesh=mesh,
        compiler_params=pltpu.CompilerParams(needs_layout_passes=False),
        out_type=[
            jax.ShapeDtypeStruct((NC, NP, D), jnp.float32),
            jax.ShapeDtypeStruct((NC, NS, NP // D, D), jnp.float32),
        ],
        scratch_types=[
            pltpu.VMEM((CPB, ECH), jnp.int32),
            pltpu.VMEM((CPB, ECH), jnp.int32),
            pltpu.VMEM((2, ECH, D), jnp.float32),
            pltpu.VMEM((NP // 32,), jnp.int32),
            pltpu.VMEM((NP // D, D), jnp.float32),
            pltpu.VMEM_SHARED((NP, D), jnp.float32),
            pltpu.SemaphoreType.DMA,
            pltpu.SemaphoreType.DMA,
            pltpu.SemaphoreType.DMA,
            pltpu.SemaphoreType.DMA,
        ],
    )
    def body(hd_hbm, nvb_hbm, src_hbm, dst_hbm, aggd_hbm, cnt_hbm,
             sidx, didx, rows2, nvb_v, cnt_v, aggd_sh, gs0, gs1, ss0, ss1):
        cid = lax.axis_index("c")
        sid = lax.axis_index("s")
        wid = sid * NC + cid
        gsem = (gs0, gs1)
        ssem = (ss0, ss1)

        pltpu.sync_copy(nvb_hbm, nvb_v)
        def zc(i, carry):
            for j in range(D // 16):
                cnt_v[i, pl.ds(j * 16, 16)] = jnp.zeros((16,), jnp.float32)
            return carry
        lax.fori_loop(0, NP // D, zc, 0)

        def zrow(i, carry):
            for j in range(D // 16):
                rows2[0, i, pl.ds(j * 16, 16)] = jnp.zeros((16,), jnp.float32)
            return carry
        lax.fori_loop(0, ECH, zrow, 0)
        for t in range(ROWS_PT // ECH):
            r0 = sid * ROWS_PT + t * ECH
            pltpu.sync_copy(rows2.at[0], aggd_sh.at[pl.ds(r0, ECH)])
        plsc.subcore_barrier()

        def _drain_scatter(k):
            pltpu.make_async_copy(hd_hbm.at[pl.ds(0, ECH)], rows2.at[k],
                                  ssem[k]).wait()

        nblk = jnp.where(cid == 0, NBLK0, NBLK1)
        cbase = jnp.where(cid == 0, sid * NCH0, NS * NCH0 + sid * NCH1)

        def block(bi, carry):
            @pl.when(bi > 0)
            def _():
                _drain_scatter(0)
                _drain_scatter(1)
            crow = cbase + bi * CPB
            pltpu.sync_copy(src_hbm.at[pl.ds(crow, CPB)], sidx)
            pltpu.sync_copy(dst_hbm.at[pl.ds(crow, CPB)], didx)

            gd = [None, None]
            sd = [None, None]
            gd[0] = pltpu.async_copy(hd_hbm.at[sidx.at[0]], rows2.at[0],
                                     gsem[0])
            for t in range(CPB):
                k = t & 1
                kn = 1 - k
                if t + 1 < CPB:
                    if sd[kn] is not None:
                        sd[kn].wait()
                    gd[kn] = pltpu.async_copy(hd_hbm.at[sidx.at[t + 1]],
                                              rows2.at[kn], gsem[kn])
                gd[k].wait()
                sd[k] = pltpu.async_copy(rows2.at[k], aggd_sh.at[didx.at[t]],
                                         ssem[k], add=True)
                for g in range(ECH // 16):
                    s16 = sidx[t, pl.ds(g * 16, 16)]
                    d16 = didx[t, pl.ds(g * 16, 16)]
                    w = plsc.load_gather(
                        nvb_v, [lax.shift_right_logical(s16, 5)])
                    bit = lax.bitwise_and(
                        lax.shift_right_logical(w, lax.bitwise_and(s16, 31)),
                        1)
                    plsc.addupdate_scatter(
                        cnt_v,
                        [lax.shift_right_logical(d16, 7),
                         lax.bitwise_and(d16, 127)],
                        bit.astype(jnp.float32))
            return carry
        lax.fori_loop(0, nblk, block, 0)

        @pl.when(nblk > 0)
        def _():
            _drain_scatter(0)
            _drain_scatter(1)
        plsc.subcore_barrier()

        pltpu.sync_copy(aggd_sh.at[pl.ds(sid * ROWS_PT, ROWS_PT)],
                        aggd_hbm.at[cid, pl.ds(sid * ROWS_PT, ROWS_PT)])
        pltpu.sync_copy(cnt_v, cnt_hbm.at[cid, sid])

    return body


def _sc_sage(hd, nvb, srcp, dstp):
    aggd, cnt = _sc_sage_kernel()(hd, nvb, srcp, dstp)
    return aggd, cnt.reshape(NW, NB, 1, BLK)


def _eye(n):
    ir = lax.broadcasted_iota(jnp.int32, (n, n), 0)
    ic = lax.broadcasted_iota(jnp.int32, (n, n), 1)
    return (ir == ic).astype(jnp.float32)


def _layerpool_body(jlo, jhi, aggd0, aggd1, cnts, hprev, wl, wr, bv, pc,
                    bcol, b2d, nvcol, nv2d,
                    hn_ref, nv_ref, gx_ref, gc_ref,
                    hp_sc, scol_sc, s2d_sc):
    p = pl.program_id(0)
    i = pl.program_id(1)
    eye = _eye(BLK)

    @pl.when(p == 0)
    def _():
        cntrow = jnp.sum(cnts[:, 0, 0, :], axis=0, keepdims=True)
        cnt = jnp.sum(jnp.broadcast_to(cntrow, (BLK, BLK)) * eye, axis=1,
                      keepdims=True)
        mean = (aggd0[...] + aggd1[...]) / jnp.maximum(cnt, 1.0)
        lin = (jnp.dot(mean, wl[...], preferred_element_type=jnp.float32)
               + jnp.dot(hprev[...], wr[...], preferred_element_type=jnp.float32)
               + bv[...])
        hp = jnp.maximum(lin, 0.0)
        hp_sc[pl.ds(i * BLK, BLK), :] = hp
        scol = jnp.tanh(jnp.dot(hp, pc[...], preferred_element_type=jnp.float32))
        scol_sc[pl.ds(i * BLK, BLK), :] = scol
        s2d_sc[pl.ds(i, 1), :] = jnp.sum(
            jnp.broadcast_to(scol, (BLK, BLK)) * eye, axis=0, keepdims=True)

    @pl.when(p == 1)
    def _():
        sic = scol_sc[pl.ds(i * BLK, BLK), :]
        hp = hp_sc[pl.ds(i * BLK, BLK), :]
        bic = bcol[...]
        idxc = i * BLK + lax.broadcasted_iota(jnp.int32, (BLK, 1), 0)
        z = jnp.zeros((BLK, 1), jnp.int32)

        def jbody(j, carry):
            rank, vcnt = carry
            srow = s2d_sc[pl.ds(j, 1), :]
            brow = b2d[pl.ds(j, 1), :]
            nvrow = nv2d[pl.ds(j, 1), :]
            beqv = (brow == bic) & (nvrow > 0.5)
            jidx = j * BLK + lax.broadcasted_iota(jnp.int32, (1, BLK), 1)
            before = (srow > sic) | ((srow == sic) & (jidx < idxc))
            dr = jnp.sum((beqv & before).astype(jnp.int32), axis=1,
                         keepdims=True)
            dv = jnp.sum(beqv.astype(jnp.int32), axis=1, keepdims=True)
            return (rank + dr, vcnt + dv)

        rank, vcnt = lax.fori_loop(jlo[0, i], jhi[0, i] + 1, jbody, (z, z))
        kk = (4 * vcnt + 4) // 5
        m = ((nvcol[...] > 0.5) & (rank < kk)).astype(jnp.float32)
        hn = hp * (m * sic)
        hn_ref[...] = hn
        nv_ref[...] = m

        giota = lax.broadcasted_iota(jnp.int32, (NG, 1), 0)
        brow_i = b2d[pl.ds(i, 1), :]
        onehot = (giota == brow_i).astype(jnp.float32)
        gx = jnp.dot(onehot, hn, preferred_element_type=jnp.float32)
        gc = jnp.dot(onehot, m, preferred_element_type=jnp.float32)

        @pl.when(i == 0)
        def _():
            gx_ref[...] = gx
            gc_ref[...] = gc

        @pl.when(i > 0)
        def _():
            gx_ref[...] += gx
            gc_ref[...] += gc


def _tc_layerpool(jlo, jhi, aggd, cnt, hprev, wl, wr, bvec, pcol,
                  batchcol, batch2d, nvcol):
    nv2d = nvcol.reshape(NB, BLK)
    full = lambda shape: pl.BlockSpec(shape, lambda p, i: (0,) * len(shape))
    blk = lambda shape: pl.BlockSpec(
        shape, lambda p, i: (i,) + (0,) * (len(shape) - 1))
    smem = pl.BlockSpec(memory_space=pltpu.SMEM)
    return pl.pallas_call(
        _layerpool_body,
        grid=(2, NB),
        in_specs=[smem, smem, blk((BLK, D)), blk((BLK, D)),
                  pl.BlockSpec((NW, 1, 1, BLK), lambda p, i: (0, i, 0, 0)),
                  blk((BLK, D)), full((D, D)), full((D, D)), full((1, D)),
                  full((D, 1)), blk((BLK, 1)), full((NB, BLK)),
                  blk((BLK, 1)), full((NB, BLK))],
        out_specs=[blk((BLK, D)), blk((BLK, 1)),
                   full((NG, D)), full((NG, 1))],
        out_shape=[jax.ShapeDtypeStruct((NP, D), jnp.float32),
                   jax.ShapeDtypeStruct((NP, 1), jnp.float32),
                   jax.ShapeDtypeStruct((NG, D), jnp.float32),
                   jax.ShapeDtypeStruct((NG, 1), jnp.float32)],
        scratch_shapes=[pltpu.VMEM((NP, D), jnp.float32),
                        pltpu.VMEM((NP, 1), jnp.float32),
                        pltpu.VMEM((NB, BLK), jnp.float32)],
    )(jlo, jhi, aggd[0], aggd[1], cnt, hprev, wl, wr, bvec, pcol,
      batchcol, batch2d, nvcol, nv2d)


def _mlp_body(gx1, gc1, gx2, gc2, gx3, gc3, w1, c1, w2, c2, w3, c3, out_ref):
    z = (gx1[...] / jnp.maximum(gc1[...], 1.0)
         + gx2[...] / jnp.maximum(gc2[...], 1.0)
         + gx3[...] / jnp.maximum(gc3[...], 1.0))
    z = jnp.maximum(jnp.dot(z, w1[...], preferred_element_type=jnp.float32) + c1[...], 0.0)
    z = jnp.maximum(jnp.dot(z, w2[...], preferred_element_type=jnp.float32) + c2[...], 0.0)
    out_ref[...] = jax.nn.sigmoid(
        jnp.dot(z, w3[...], preferred_element_type=jnp.float32) + c3[...])


def _tc_mlp(g, w1, c1, w2, c2, w3, c3):
    return pl.pallas_call(
        _mlp_body,
        out_shape=jax.ShapeDtypeStruct((NG, 1), jnp.float32),
    )(g[0][0], g[0][1], g[1][0], g[1][1], g[2][0], g[2][1],
      w1, c1, w2, c2, w3, c3)


def kernel(x, edge_index, batch, emb_table, Wl1, Wr1, b1, p1, Wl2, Wr2, b2,
           p2, Wl3, Wr3, b3, p3, W1, bl1, W2, bl2, W3, bl3):
    f32 = jnp.float32
    xi = jnp.concatenate([x[:, 0].astype(jnp.int32),
                          jnp.zeros((NP - N,), jnp.int32)])
    epad = jnp.full((EPAD - E,), N, jnp.int32)
    srcp = jnp.concatenate([edge_index[0].astype(jnp.int32), epad]).reshape(
        EPAD // ECH, ECH)
    dstp = jnp.concatenate([edge_index[1].astype(jnp.int32), epad]).reshape(
        EPAD // ECH, ECH)
    bitw = jnp.left_shift(jnp.int32(1), jnp.arange(32, dtype=jnp.int32))
    batchp = jnp.concatenate([batch.astype(jnp.int32),
                              jnp.full((NP - N,), NG, jnp.int32)])
    batchcol = batchp.reshape(NP, 1)
    batch2d = batchp.reshape(NB, BLK)

    blo = jnp.min(batch2d, axis=1)
    bhi = jnp.max(batch2d, axis=1)
    ov = (blo[None, :] <= bhi[:, None]) & (bhi[None, :] >= blo[:, None])
    jlo = jnp.argmax(ov, axis=1).astype(jnp.int32).reshape(1, NB)
    jhi = (NB - 1 - jnp.argmax(ov[:, ::-1], axis=1)).astype(jnp.int32).reshape(1, NB)

    nvcol = (jnp.arange(NP) < N).astype(f32).reshape(NP, 1)
    hd = _sc_embed(emb_table, xi)

    layers = ((Wl1, Wr1, b1, p1), (Wl2, Wr2, b2, p2), (Wl3, Wr3, b3, p3))
    gaps = []
    for wl, wr, bb, pp in layers:
        nvb = jnp.sum(
            jnp.where(nvcol.reshape(NP // 32, 32) > 0.5, bitw, 0)
            .astype(jnp.int32), axis=1).astype(jnp.int32)
        aggd, cnt = _sc_sage(hd, nvb, srcp, dstp)
        pcol = (pp / jnp.linalg.norm(pp)).reshape(D, 1)
        hd, nvcol, gx, gc = _tc_layerpool(jlo, jhi, aggd, cnt, hd, wl, wr,
                                          bb.reshape(1, D), pcol, batchcol,
                                          batch2d, nvcol)
        gaps.append((gx, gc))

    out = _tc_mlp(gaps, W1, bl1.reshape(1, 128), W2, bl2.reshape(1, 64),
                  W3, bl3.reshape(1, 1))
    return out[:, 0]


# speedup vs baseline: 1.0053x; 1.0053x over previous
import functools

import jax
import jax.numpy as jnp
from jax import lax
from jax.experimental import pallas as pl
from jax.experimental.pallas import tpu as pltpu
from jax.experimental.pallas import tpu_sc as plsc

N = 10000
NP = 10240
E = 320000
D = 128
VOCAB = 100010
NG = 512
BLK = 256
NB = NP // BLK
NC = 2
NS = 16
NW = NC * NS
ECH = 128
CPB = 8
NBLK0 = 19
NBLK1 = 1
NCH0 = CPB * NBLK0
NCH1 = CPB * NBLK1
EPAD = ECH * CPB * (NBLK0 + NBLK1) * NS
ROWS_PT = NP // NS


@functools.cache
def _sc_embed_kernel():
    mesh = plsc.VectorSubcoreMesh(core_axis_name="c", subcore_axis_name="s")

    @functools.partial(
        pl.kernel, mesh=mesh,
        compiler_params=pltpu.CompilerParams(needs_layout_passes=False),
        out_type=jax.ShapeDtypeStruct((NP, D), jnp.float32),
        scratch_types=[
            pltpu.VMEM((1, 64), jnp.int32),
            pltpu.VMEM((64, D), jnp.float32),
            pltpu.SemaphoreType.DMA,
        ],
    )
    def body(emb_hbm, xi_hbm, out_hbm, eidx, erows, sem):
        cid = lax.axis_index("c")
        sid = lax.axis_index("s")
        wid = sid * NC + cid
        def chunk(t, carry):
            base = wid * (NP // NW) + t * 64
            pltpu.sync_copy(xi_hbm.at[pl.ds(base, 64)], eidx.at[0])
            pltpu.async_copy(emb_hbm.at[eidx.at[0]], erows, sem).wait()
            pltpu.sync_copy(erows, out_hbm.at[pl.ds(base, 64)])
            return carry
        lax.fori_loop(0, (NP // NW) // 64, chunk, 0)

    return body


def _sc_embed(emb, xi):
    return _sc_embed_kernel()(emb, xi)


@functools.cache
def _sc_sage_kernel():
    mesh = plsc.VectorSubcoreMesh(core_axis_name="c", subcore_axis_name="s")

    @functools.partial(
        pl.kernel, mesh=mesh,
        compiler_params=pltpu.CompilerParams(needs_layout_passes=False),
        out_type=[
            jax.ShapeDtypeStruct((NC, NP, D), jnp.float32),
            jax.ShapeDtypeStruct((NC, NS, NP // D, D), jnp.float32),
        ],
        scratch_types=[
            pltpu.VMEM((CPB, ECH), jnp.int32),
            pltpu.VMEM((CPB, ECH), jnp.int32),
            pltpu.VMEM((2, ECH, D), jnp.float32),
            pltpu.VMEM((NP // 32,), jnp.int32),
            pltpu.VMEM((NP // D, D), jnp.float32),
            pltpu.VMEM_SHARED((NP, D), jnp.float32),
            pltpu.SemaphoreType.DMA,
            pltpu.SemaphoreType.DMA,
            pltpu.SemaphoreType.DMA,
            pltpu.SemaphoreType.DMA,
        ],
    )
    def body(hd_hbm, nvb_hbm, src_hbm, dst_hbm, aggd_hbm, cnt_hbm,
             sidx, didx, rows2, nvb_v, cnt_v, aggd_sh, gs0, gs1, ss0, ss1):
        cid = lax.axis_index("c")
        sid = lax.axis_index("s")
        wid = sid * NC + cid
        gsem = (gs0, gs1)
        ssem = (ss0, ss1)

        pltpu.sync_copy(nvb_hbm, nvb_v)
        def zc(i, carry):
            for j in range(D // 16):
                cnt_v[i, pl.ds(j * 16, 16)] = jnp.zeros((16,), jnp.float32)
            return carry
        lax.fori_loop(0, NP // D, zc, 0)

        def zrow(i, carry):
            for j in range(D // 16):
                rows2[0, i, pl.ds(j * 16, 16)] = jnp.zeros((16,), jnp.float32)
            return carry
        lax.fori_loop(0, ECH, zrow, 0)
        for t in range(ROWS_PT // ECH):
            r0 = sid * ROWS_PT + t * ECH
            pltpu.sync_copy(rows2.at[0], aggd_sh.at[pl.ds(r0, ECH)])
        plsc.subcore_barrier()

        def _drain_scatter(k):
            pltpu.make_async_copy(hd_hbm.at[pl.ds(0, ECH)], rows2.at[k],
                                  ssem[k]).wait()

        nblk = jnp.where(cid == 0, NBLK0, NBLK1)
        cbase = jnp.where(cid == 0, sid * NCH0, NS * NCH0 + sid * NCH1)

        def block(bi, carry):
            @pl.when(bi > 0)
            def _():
                _drain_scatter(0)
                _drain_scatter(1)
            crow = cbase + bi * CPB
            pltpu.sync_copy(src_hbm.at[pl.ds(crow, CPB)], sidx)
            pltpu.sync_copy(dst_hbm.at[pl.ds(crow, CPB)], didx)

            gd = [None, None]
            sd = [None, None]
            gd[0] = pltpu.async_copy(hd_hbm.at[sidx.at[0]], rows2.at[0],
                                     gsem[0])
            for t in range(CPB):
                k = t & 1
                kn = 1 - k
                if t + 1 < CPB:
                    if sd[kn] is not None:
                        sd[kn].wait()
                    gd[kn] = pltpu.async_copy(hd_hbm.at[sidx.at[t + 1]],
                                              rows2.at[kn], gsem[kn])
                gd[k].wait()
                sd[k] = pltpu.async_copy(rows2.at[k], aggd_sh.at[didx.at[t]],
                                         ssem[k], add=True)
                for g in range(ECH // 16):
                    s16 = sidx[t, pl.ds(g * 16, 16)]
                    d16 = didx[t, pl.ds(g * 16, 16)]
                    w = plsc.load_gather(
                        nvb_v, [lax.shift_right_logical(s16, 5)])
                    bit = lax.bitwise_and(
                        lax.shift_right_logical(w, lax.bitwise_and(s16, 31)),
                        1)
                    plsc.addupdate_scatter(
                        cnt_v,
                        [lax.shift_right_logical(d16, 7),
                         lax.bitwise_and(d16, 127)],
                        bit.astype(jnp.float32))
            return carry
        lax.fori_loop(0, nblk, block, 0)

        @pl.when(nblk > 0)
        def _():
            _drain_scatter(0)
            _drain_scatter(1)
        plsc.subcore_barrier()

        pltpu.sync_copy(aggd_sh.at[pl.ds(sid * ROWS_PT, ROWS_PT)],
                        aggd_hbm.at[cid, pl.ds(sid * ROWS_PT, ROWS_PT)])
        pltpu.sync_copy(cnt_v, cnt_hbm.at[cid, sid])

    return body


def _sc_sage(hd, nvb, srcp, dstp):
    aggd, cnt = _sc_sage_kernel()(hd, nvb, srcp, dstp)
    return aggd, cnt.reshape(NW, NB, 1, BLK)


def _eye(n):
    ir = lax.broadcasted_iota(jnp.int32, (n, n), 0)
    ic = lax.broadcasted_iota(jnp.int32, (n, n), 1)
    return (ir == ic).astype(jnp.float32)


def _layer_body(aggd0, aggd1, cnts, hprev, wl, wr, bv, pc, hp_ref, sc_ref):
    cntrow = jnp.sum(cnts[:, 0, 0, :], axis=0, keepdims=True)
    eye = _eye(BLK)
    cnt = jnp.sum(jnp.broadcast_to(cntrow, (BLK, BLK)) * eye, axis=1,
                  keepdims=True)
    mean = (aggd0[...] + aggd1[...]) / jnp.maximum(cnt, 1.0)
    lin = (jnp.dot(mean, wl[...], preferred_element_type=jnp.float32)
           + jnp.dot(hprev[...], wr[...], preferred_element_type=jnp.float32)
           + bv[...])
    hp = jnp.maximum(lin, 0.0)
    hp_ref[...] = hp
    sc_ref[...] = jnp.tanh(jnp.dot(hp, pc[...], preferred_element_type=jnp.float32))


def _tc_layer(aggd, cnt, hprev, wl, wr, bvec, pcol):
    full = lambda shape: pl.BlockSpec(shape, lambda i: (0,) * len(shape))
    blk = lambda shape: pl.BlockSpec(shape, lambda i: (i,) + (0,) * (len(shape) - 1))
    return pl.pallas_call(
        _layer_body,
        grid=(NB,),
        in_specs=[blk((BLK, D)), blk((BLK, D)),
                  pl.BlockSpec((NW, 1, 1, BLK), lambda i: (0, i, 0, 0)),
                  blk((BLK, D)), full((D, D)), full((D, D)), full((1, D)),
                  full((D, 1))],
        out_specs=[blk((BLK, D)), blk((BLK, 1))],
        out_shape=[jax.ShapeDtypeStruct((NP, D), jnp.float32),
                   jax.ShapeDtypeStruct((NP, 1), jnp.float32)],
    )(aggd[0], aggd[1], cnt, hprev, wl, wr, bvec, pcol)


def _pool_body(jlo, jhi, scol, s2d, bcol, b2d, nvcol, nv2d, hp,
               hn_ref, nv_ref, gx_ref, gc_ref):
    i = pl.program_id(0)
    sic = scol[...]
    bic = bcol[...]
    idxc = i * BLK + lax.broadcasted_iota(jnp.int32, (BLK, 1), 0)
    z = jnp.zeros((BLK, 1), jnp.int32)

    def jbody(j, carry):
        rank, vcnt = carry
        srow = s2d[pl.ds(j, 1), :]
        brow = b2d[pl.ds(j, 1), :]
        nvrow = nv2d[pl.ds(j, 1), :]
        beqv = (brow == bic) & (nvrow > 0.5)
        jidx = j * BLK + lax.broadcasted_iota(jnp.int32, (1, BLK), 1)
        before = (srow > sic) | ((srow == sic) & (jidx < idxc))
        dr = jnp.sum((beqv & before).astype(jnp.int32), axis=1, keepdims=True)
        dv = jnp.sum(beqv.astype(jnp.int32), axis=1, keepdims=True)
        return (rank + dr, vcnt + dv)

    rank, vcnt = lax.fori_loop(jlo[0, i], jhi[0, i] + 1, jbody, (z, z))
    kk = (4 * vcnt + 4) // 5
    m = ((nvcol[...] > 0.5) & (rank < kk)).astype(jnp.float32)
    hn = hp[...] * (m * sic)
    hn_ref[...] = hn
    nv_ref[...] = m

    giota = lax.broadcasted_iota(jnp.int32, (NG, 1), 0)
    brow_i = b2d[pl.ds(i, 1), :]
    onehot = (giota == brow_i).astype(jnp.float32)
    gx = jnp.dot(onehot, hn, preferred_element_type=jnp.float32)
    gc = jnp.dot(onehot, m, preferred_element_type=jnp.float32)

    @pl.when(i == 0)
    def _():
        gx_ref[...] = gx
        gc_ref[...] = gc

    @pl.when(i > 0)
    def _():
        gx_ref[...] += gx
        gc_ref[...] += gc


def _tc_pool(jlo, jhi, scol, batchcol, batch2d, nvcol, hp):
    s2d = scol.reshape(NB, BLK)
    nv2d = nvcol.reshape(NB, BLK)
    full = lambda shape: pl.BlockSpec(shape, lambda i: (0,) * len(shape))
    blk = lambda shape: pl.BlockSpec(shape, lambda i: (i,) + (0,) * (len(shape) - 1))
    smem = pl.BlockSpec(memory_space=pltpu.SMEM)
    return pl.pallas_call(
        _pool_body,
        grid=(NB,),
        in_specs=[smem, smem, blk((BLK, 1)), full((NB, BLK)), blk((BLK, 1)),
                  full((NB, BLK)), blk((BLK, 1)), full((NB, BLK)),
                  blk((BLK, D))],
        out_specs=[blk((BLK, D)), blk((BLK, 1)),
                   full((NG, D)), full((NG, 1))],
        out_shape=[jax.ShapeDtypeStruct((NP, D), jnp.float32),
                   jax.ShapeDtypeStruct((NP, 1), jnp.float32),
                   jax.ShapeDtypeStruct((NG, D), jnp.float32),
                   jax.ShapeDtypeStruct((NG, 1), jnp.float32)],
    )(jlo, jhi, scol, s2d, batchcol, batch2d, nvcol, nv2d, hp)


def _mlp_body(gx1, gc1, gx2, gc2, gx3, gc3, w1, c1, w2, c2, w3, c3, out_ref):
    z = (gx1[...] / jnp.maximum(gc1[...], 1.0)
         + gx2[...] / jnp.maximum(gc2[...], 1.0)
         + gx3[...] / jnp.maximum(gc3[...], 1.0))
    z = jnp.maximum(jnp.dot(z, w1[...], preferred_element_type=jnp.float32) + c1[...], 0.0)
    z = jnp.maximum(jnp.dot(z, w2[...], preferred_element_type=jnp.float32) + c2[...], 0.0)
    out_ref[...] = jax.nn.sigmoid(
        jnp.dot(z, w3[...], preferred_element_type=jnp.float32) + c3[...])


def _tc_mlp(g, w1, c1, w2, c2, w3, c3):
    return pl.pallas_call(
        _mlp_body,
        out_shape=jax.ShapeDtypeStruct((NG, 1), jnp.float32),
    )(g[0][0], g[0][1], g[1][0], g[1][1], g[2][0], g[2][1],
      w1, c1, w2, c2, w3, c3)


def kernel(x, edge_index, batch, emb_table, Wl1, Wr1, b1, p1, Wl2, Wr2, b2,
           p2, Wl3, Wr3, b3, p3, W1, bl1, W2, bl2, W3, bl3):
    f32 = jnp.float32
    xi = jnp.concatenate([x[:, 0].astype(jnp.int32),
                          jnp.zeros((NP - N,), jnp.int32)])
    epad = jnp.full((EPAD - E,), N, jnp.int32)
    srcp = jnp.concatenate([edge_index[0].astype(jnp.int32), epad]).reshape(
        EPAD // ECH, ECH)
    dstp = jnp.concatenate([edge_index[1].astype(jnp.int32), epad]).reshape(
        EPAD // ECH, ECH)
    bitw = jnp.left_shift(jnp.int32(1), jnp.arange(32, dtype=jnp.int32))
    batchp = jnp.concatenate([batch.astype(jnp.int32),
                              jnp.full((NP - N,), NG, jnp.int32)])
    batchcol = batchp.reshape(NP, 1)
    batch2d = batchp.reshape(NB, BLK)

    blo = jnp.min(batch2d, axis=1)
    bhi = jnp.max(batch2d, axis=1)
    ov = (blo[None, :] <= bhi[:, None]) & (bhi[None, :] >= blo[:, None])
    jlo = jnp.argmax(ov, axis=1).astype(jnp.int32).reshape(1, NB)
    jhi = (NB - 1 - jnp.argmax(ov[:, ::-1], axis=1)).astype(jnp.int32).reshape(1, NB)

    nvcol = (jnp.arange(NP) < N).astype(f32).reshape(NP, 1)
    hd = _sc_embed(emb_table, xi)

    layers = ((Wl1, Wr1, b1, p1), (Wl2, Wr2, b2, p2), (Wl3, Wr3, b3, p3))
    gaps = []
    for wl, wr, bb, pp in layers:
        nvb = jnp.sum(
            jnp.where(nvcol.reshape(NP // 32, 32) > 0.5, bitw, 0)
            .astype(jnp.int32), axis=1).astype(jnp.int32)
        aggd, cnt = _sc_sage(hd, nvb, srcp, dstp)
        pcol = (pp / jnp.linalg.norm(pp)).reshape(D, 1)
        hp, scol = _tc_layer(aggd, cnt, hd, wl, wr, bb.reshape(1, D), pcol)
        hd, nvcol, gx, gc = _tc_pool(jlo, jhi, scol, batchcol, batch2d,
                                     nvcol, hp)
        gaps.append((gx, gc))

    out = _tc_mlp(gaps, W1, bl1.reshape(1, 128), W2, bl2.reshape(1, 64),
                  W3, bl3.reshape(1, 1))
    return out[:, 0]
